# Initial kernel scaffold; baseline (speedup 1.0000x reference)
import jax
import jax.numpy as jnp
from jax.experimental import pallas as pl


def kernel(x, v, v_mask, log_epsi, nv_W1, nv_b1, nv_W2, nv_b2, nv_W3, nv_b3, nv_W4, nv_b4, nv_W5, nv_b5, nr_W1, nr_b1, nr_W2, nr_b2, nr_W3, nr_b3, nr_W4, nr_b4, nr_W5, nr_b5):
    raise NotImplementedError("write your pallas kernel here")


# speedup vs baseline: 1.3627x; 1.3627x over previous
import jax
import jax.numpy as jnp
from jax.experimental import pallas as pl
from jax.experimental.pallas import tpu as pltpu

C = 768
DIM = 768
NSTEPS = 2
BLK = 256


def _step_kernel(x_ref, v_ref, sldj_ref, le_ref, vm_ref,
                 rw1x, rw1v, rb1, rw2, rb2, rw3, rb3, rw4, rb4, rw5, rb5,
                 vw1x, vw1v, vw1g, vb1, vw2, vb2, vw3, vb3, vw4, vb4,
                 vw5s, vw5q, vw5t, vb5s, vb5q, vb5t,
                 v_out, sldj_out):
    x = x_ref[...]
    v = v_ref[...]
    vm = vm_ref[...]
    vmc = 1.0 - vm
    epsi = jnp.exp(le_ref[0, 0]) / (2.0 * NSTEPS)

    def dot(a, b):
        return jax.lax.dot_general(a, b, (((1,), (0,)), ((), ())),
                                   preferred_element_type=jnp.float32)

    def half(v_in, m_act, m_upd):
        va = m_act * v_in
        h = jax.nn.relu(dot(x, rw1x[...]) + dot(va, rw1v[...]) + rb1[...])
        h = jax.nn.relu(dot(h, rw2[...]) + rb2[...])
        h = jax.nn.relu(dot(h, rw3[...]) + rb3[...])
        h = jax.nn.relu(dot(h, rw4[...]) + rb4[...])
        grad_e = dot(h, rw5[...]) + rb5[...]
        g = jax.nn.relu(dot(x, vw1x[...]) + dot(va, vw1v[...])
                        + dot(grad_e, vw1g[...]) + vb1[...])
        g = jax.nn.relu(dot(g, vw2[...]) + vb2[...])
        g = jax.nn.relu(dot(g, vw3[...]) + vb3[...])
        g = jnp.tanh(dot(g, vw4[...]) + vb4[...])
        s = dot(g, vw5s[...]) + vb5s[...]
        q = dot(g, vw5q[...]) + vb5q[...]
        t = dot(g, vw5t[...]) + vb5t[...]
        v_new = va + m_upd * (v_in * jnp.exp(s)
                              - epsi * (grad_e * jnp.exp(q) + t))
        dsldj = jnp.sum(m_upd * s, axis=1, keepdims=True)
        return v_new, dsldj

    v1, d1 = half(v, vm, vmc)
    v2, d2 = half(v1, vmc, vm)
    v_out[...] = v2
    sldj_out[...] = sldj_ref[...] + d1 + d2


def _run_step(x, v, sldj, le, vm, weights):
    b = x.shape[0]
    grid = (b // BLK,)

    def batch_spec(cols):
        return pl.BlockSpec((BLK, cols), lambda i: (i, 0))

    def full_spec(arr):
        return pl.BlockSpec(arr.shape, lambda i: (0,) * arr.ndim)

    in_specs = ([batch_spec(C), batch_spec(C), batch_spec(1),
                 full_spec(le), full_spec(vm)]
                + [full_spec(w) for w in weights])
    out_specs = [batch_spec(C), batch_spec(1)]
    out_shape = [jax.ShapeDtypeStruct((b, C), jnp.float32),
                 jax.ShapeDtypeStruct((b, 1), jnp.float32)]
    return pl.pallas_call(
        _step_kernel,
        grid=grid,
        in_specs=in_specs,
        out_specs=out_specs,
        out_shape=out_shape,
        compiler_params=pltpu.CompilerParams(
            vmem_limit_bytes=62 * 1024 * 1024),
    )(x, v, sldj, le, vm, *weights)


def kernel(x, v, v_mask, log_epsi,
           nv_W1, nv_b1, nv_W2, nv_b2, nv_W3, nv_b3, nv_W4, nv_b4,
           nv_W5, nv_b5,
           nr_W1, nr_b1, nr_W2, nr_b2, nr_W3, nr_b3, nr_W4, nr_b4,
           nr_W5, nr_b5):
    b = x.shape[0]
    le = log_epsi.reshape(1, 1).astype(jnp.float32)
    sldj = jnp.zeros((b, 1), dtype=jnp.float32)

    shared = dict(
        rw2=nr_W2.T, rb2=nr_b2.reshape(1, DIM),
        rw3=nr_W3.T, rb3=nr_b3.reshape(1, DIM),
        rw4=nr_W4.T, rb4=nr_b4.reshape(1, DIM),
        vw2=nv_W2.T, vb2=nv_b2.reshape(1, DIM),
        vw3=nv_W3.T, vb3=nv_b3.reshape(1, DIM),
        vw4=nv_W4.T, vb4=nv_b4.reshape(1, DIM),
    )

    for i in range(NSTEPS):
        w = shared
        weights = [
            nr_W1[i, :, :C].T, nr_W1[i, :, C:].T, nr_b1[i].reshape(1, DIM),
            w["rw2"], w["rb2"], w["rw3"], w["rb3"], w["rw4"], w["rb4"],
            nr_W5[i].T, nr_b5[i].reshape(1, C),
            nv_W1[i, :, :C].T, nv_W1[i, :, C:2 * C].T, nv_W1[i, :, 2 * C:].T,
            nv_b1[i].reshape(1, DIM),
            w["vw2"], w["vb2"], w["vw3"], w["vb3"], w["vw4"], w["vb4"],
            nv_W5[i, :C, :].T, nv_W5[i, C:2 * C, :].T, nv_W5[i, 2 * C:, :].T,
            nv_b5[i, :C].reshape(1, C), nv_b5[i, C:2 * C].reshape(1, C),
            nv_b5[i, 2 * C:].reshape(1, C),
        ]
        vm = v_mask[i].reshape(1, C)
        v, sldj = _run_step(x, v, sldj, le, vm, weights)

    return v, sldj.reshape(b)


# speedup vs baseline: 1.4329x; 1.0516x over previous
import jax
import jax.numpy as jnp
from jax.experimental import pallas as pl
from jax.experimental.pallas import tpu as pltpu

C = 768
DIM = 768
NSTEPS = 2
BLK = 256


def _flow_kernel(x_ref, v_ref, le_ref, vm_ref,
                 rw1x, rw1v, rb1, rw2, rb2, rw3, rb3, rw4, rb4, rw5, rb5,
                 vw1x, vw1v, vw1g, vb1, vw2, vb2, vw3, vb3, vw4, vb4,
                 vw5s, vw5q, vw5t, vb5s, vb5q, vb5t,
                 v_out, sldj_out):
    x = x_ref[...]
    v = v_ref[...]
    xb = x.astype(jnp.bfloat16)
    epsi = jnp.exp(le_ref[0, 0]) / (2.0 * NSTEPS)

    def fdot(a, b):
        return jax.lax.dot_general(a, b, (((1,), (0,)), ((), ())),
                                   preferred_element_type=jnp.float32)

    def bdot(a, b):
        return fdot(a.astype(jnp.bfloat16), b)

    def half(i, v_in, m_act, m_upd):
        va = m_act * v_in
        vab = va.astype(jnp.bfloat16)
        h = jax.nn.relu(fdot(xb, rw1x[i]) + fdot(vab, rw1v[i]) + rb1[i])
        h = jax.nn.relu(bdot(h, rw2[...]) + rb2[...])
        h = jax.nn.relu(bdot(h, rw3[...]) + rb3[...])
        h = jax.nn.relu(bdot(h, rw4[...]) + rb4[...])
        grad_e = bdot(h, rw5[i]) + rb5[i]
        g = jax.nn.relu(fdot(x, vw1x[i]) + fdot(va, vw1v[i])
                        + fdot(grad_e, vw1g[i]) + vb1[i])
        g = jax.nn.relu(bdot(g, vw2[...]) + vb2[...])
        g = jax.nn.relu(bdot(g, vw3[...]) + vb3[...])
        g = jnp.tanh(fdot(g, vw4[...]) + vb4[...])
        s = fdot(g, vw5s[i]) + vb5s[i]
        q = bdot(g, vw5q[i]) + vb5q[i]
        t = bdot(g, vw5t[i]) + vb5t[i]
        v_new = va + m_upd * (v_in * jnp.exp(s)
                              - epsi * (grad_e * jnp.exp(q) + t))
        dsldj = jnp.sum(m_upd * s, axis=1, keepdims=True)
        return v_new, dsldj

    sldj = jnp.zeros((v.shape[0], 1), dtype=jnp.float32)
    for i in range(NSTEPS):
        vm = vm_ref[i]
        vmc = 1.0 - vm
        v, d = half(i, v, vm, vmc)
        sldj = sldj + d
        v, d = half(i, v, vmc, vm)
        sldj = sldj + d
    v_out[...] = v
    sldj_out[...] = sldj


def kernel(x, v, v_mask, log_epsi,
           nv_W1, nv_b1, nv_W2, nv_b2, nv_W3, nv_b3, nv_W4, nv_b4,
           nv_W5, nv_b5,
           nr_W1, nr_b1, nr_W2, nr_b2, nr_W3, nr_b3, nr_W4, nr_b4,
           nr_W5, nr_b5):
    b = x.shape[0]
    f32 = jnp.float32
    bf16 = jnp.bfloat16
    n = NSTEPS

    def t_steps(w, dtype):
        return jnp.swapaxes(w[:n], 1, 2).astype(dtype)

    weights = [
        t_steps(nr_W1[:, :, :C], bf16), t_steps(nr_W1[:, :, C:], bf16),
        nr_b1[:n].reshape(n, 1, DIM),
        nr_W2.T.astype(bf16), nr_b2.reshape(1, DIM),
        nr_W3.T.astype(bf16), nr_b3.reshape(1, DIM),
        nr_W4.T.astype(bf16), nr_b4.reshape(1, DIM),
        t_steps(nr_W5, bf16), nr_b5[:n].reshape(n, 1, C),
        t_steps(nv_W1[:, :, :C], f32), t_steps(nv_W1[:, :, C:2 * C], f32),
        t_steps(nv_W1[:, :, 2 * C:], f32), nv_b1[:n].reshape(n, 1, DIM),
        nv_W2.T.astype(bf16), nv_b2.reshape(1, DIM),
        nv_W3.T.astype(bf16), nv_b3.reshape(1, DIM),
        nv_W4.T.astype(f32), nv_b4.reshape(1, DIM),
        t_steps(nv_W5[:, :C, :], f32), t_steps(nv_W5[:, C:2 * C, :], bf16),
        t_steps(nv_W5[:, 2 * C:, :], bf16),
        nv_b5[:n, :C].reshape(n, 1, C), nv_b5[:n, C:2 * C].reshape(n, 1, C),
        nv_b5[:n, 2 * C:].reshape(n, 1, C),
    ]
    le = log_epsi.reshape(1, 1).astype(f32)
    vm = v_mask[:n].reshape(n, 1, C).astype(f32)

    def batch_spec(cols):
        return pl.BlockSpec((BLK, cols), lambda i: (i, 0))

    def full_spec(arr):
        return pl.BlockSpec(arr.shape, lambda i: (0,) * arr.ndim)

    in_specs = ([batch_spec(C), batch_spec(C), full_spec(le), full_spec(vm)]
                + [full_spec(w) for w in weights])
    v_out, sldj = pl.pallas_call(
        _flow_kernel,
        grid=(b // BLK,),
        in_specs=in_specs,
        out_specs=[batch_spec(C), batch_spec(1)],
        out_shape=[jax.ShapeDtypeStruct((b, C), f32),
                   jax.ShapeDtypeStruct((b, 1), f32)],
        compiler_params=pltpu.CompilerParams(
            vmem_limit_bytes=62 * 1024 * 1024),
    )(x, v, le, vm, *weights)
    return v_out, sldj.reshape(b)


# speedup vs baseline: 1.9478x; 1.3593x over previous
import jax
import jax.numpy as jnp
from jax.experimental import pallas as pl
from jax.experimental.pallas import tpu as pltpu

C = 768
DIM = 768
NSTEPS = 2
BLK = 256


def _flow_kernel(x_ref, v_ref, le_ref, vm_ref,
                 rw1, rb1, rw2, rb2, rw3, rb3, rw4, rb4, rw5, rb5,
                 vw1, vb1, vw2, vb2, vw3, vb3, vw4, vb4, vw5, vb5,
                 v_out, sldj_out):
    x = x_ref[...]
    v = v_ref[...]
    xb = x.astype(jnp.bfloat16)
    epsi = jnp.exp(le_ref[0, 0]) / (2.0 * NSTEPS)

    def dot_t(a, b):
        return jax.lax.dot_general(a.astype(jnp.bfloat16), b,
                                   (((1,), (1,)), ((), ())),
                                   preferred_element_type=jnp.float32)

    def half(i, v_in, m_act, m_upd):
        va = m_act * v_in
        vab = va.astype(jnp.bfloat16)
        xin = jnp.concatenate([xb, vab], axis=1)
        h = jax.nn.relu(dot_t(xin, rw1[i]) + rb1[i])
        h = jax.nn.relu(dot_t(h, rw2[...]) + rb2[...])
        h = jax.nn.relu(dot_t(h, rw3[...]) + rb3[...])
        h = jax.nn.relu(dot_t(h, rw4[...]) + rb4[...])
        grad_e = dot_t(h, rw5[i]) + rb5[i]
        xin2 = jnp.concatenate([xb, vab, grad_e.astype(jnp.bfloat16)],
                               axis=1)
        g = jax.nn.relu(dot_t(xin2, vw1[i]) + vb1[i])
        g = jax.nn.relu(dot_t(g, vw2[...]) + vb2[...])
        g = jax.nn.relu(dot_t(g, vw3[...]) + vb3[...])
        g = jnp.tanh(dot_t(g, vw4[...]) + vb4[...])
        sqt = dot_t(g, vw5[i]) + vb5[i]
        s = sqt[:, :C]
        q = sqt[:, C:2 * C]
        t = sqt[:, 2 * C:]
        v_new = va + m_upd * (v_in * jnp.exp(s)
                              - epsi * (grad_e * jnp.exp(q) + t))
        dsldj = jnp.sum(m_upd * s, axis=1, keepdims=True)
        return v_new, dsldj

    sldj = jnp.zeros((v.shape[0], 1), dtype=jnp.float32)
    for i in range(NSTEPS):
        vm = vm_ref[i]
        vmc = 1.0 - vm
        v, d = half(i, v, vm, vmc)
        sldj = sldj + d
        v, d = half(i, v, vmc, vm)
        sldj = sldj + d
    v_out[...] = v
    sldj_out[...] = sldj


def kernel(x, v, v_mask, log_epsi,
           nv_W1, nv_b1, nv_W2, nv_b2, nv_W3, nv_b3, nv_W4, nv_b4,
           nv_W5, nv_b5,
           nr_W1, nr_b1, nr_W2, nr_b2, nr_W3, nr_b3, nr_W4, nr_b4,
           nr_W5, nr_b5):
    b = x.shape[0]
    f32 = jnp.float32
    bf16 = jnp.bfloat16
    n = NSTEPS

    weights = [
        nr_W1.astype(bf16), nr_b1.reshape(n, 1, DIM),
        nr_W2.astype(bf16), nr_b2.reshape(1, DIM),
        nr_W3.astype(bf16), nr_b3.reshape(1, DIM),
        nr_W4.astype(bf16), nr_b4.reshape(1, DIM),
        nr_W5.astype(bf16), nr_b5.reshape(n, 1, C),
        nv_W1.astype(bf16), nv_b1.reshape(n + 1, 1, DIM),
        nv_W2.astype(bf16), nv_b2.reshape(1, DIM),
        nv_W3.astype(bf16), nv_b3.reshape(1, DIM),
        nv_W4.astype(bf16), nv_b4.reshape(1, DIM),
        nv_W5.astype(bf16), nv_b5.reshape(n + 1, 1, 3 * C),
    ]
    le = log_epsi.reshape(1, 1).astype(f32)
    vm = v_mask.reshape(n, 1, C).astype(f32)

    def batch_spec(cols):
        return pl.BlockSpec((BLK, cols), lambda i: (i, 0))

    def head_spec(arr):
        if arr.ndim == 3 and arr.shape[0] > n:
            shape = (n,) + arr.shape[1:]
        else:
            shape = arr.shape
        return pl.BlockSpec(shape, lambda i: (0,) * arr.ndim)

    in_specs = ([batch_spec(C), batch_spec(C),
                 pl.BlockSpec(le.shape, lambda i: (0, 0)),
                 pl.BlockSpec(vm.shape, lambda i: (0, 0, 0))]
                + [head_spec(w) for w in weights])
    v_out, sldj = pl.pallas_call(
        _flow_kernel,
        grid=(b // BLK,),
        in_specs=in_specs,
        out_specs=[batch_spec(C), batch_spec(1)],
        out_shape=[jax.ShapeDtypeStruct((b, C), f32),
                   jax.ShapeDtypeStruct((b, 1), f32)],
        compiler_params=pltpu.CompilerParams(
            vmem_limit_bytes=62 * 1024 * 1024),
    )(x, v, le, vm, *weights)
    return v_out, sldj.reshape(b)
